# baseline (speedup 1.0000x reference)
import functools

import jax
import jax.numpy as jnp
from jax import lax
from jax.experimental import pallas as pl
from jax.experimental.pallas import tpu as pltpu
from jax.experimental.pallas import tpu_sc as plsc

N = 10000
E = 320000
IN_SIZE = 128
HID = 16
HEADS = 8
OUT_SIZE = 16

NP = 10112
K = 128
NCHUNKP = 2592
EPAD = NCHUNKP * K
CPW1 = NCHUNKP // 16
CPW2 = NCHUNKP // 32
D1 = 80
D2 = 32
NPAD2 = 10016
TAB2 = 10016

_SC_PARAMS = pltpu.CompilerParams(
    needs_layout_passes=False, use_tc_tiling_on_sc=False)
A1 = 72


def _ring(idx_hbm, gathers, acc_sh, idx_b, stage_b,
          isems, cpw, row_of, compute):
  def issue_idx(it, b):
    return pltpu.async_copy(idx_hbm.at[row_of(it)], idx_b[b], isems[b])

  def issue_gather(b):
    for src_hbm, idxrow, bufs, sems in gathers:
      pltpu.async_copy(src_hbm.at[idx_b[b].at[idxrow]], bufs[b], sems[b])

  def wait_idx(b):
    pltpu.make_async_copy(idx_hbm.at[0], idx_b[b], isems[b]).wait()

  def wait_gather(b):
    for src_hbm, idxrow, bufs, sems in gathers:
      pltpu.make_async_copy(
          src_hbm.at[idx_b[b].at[idxrow]], bufs[b], sems[b]).wait()

  pltpu.sync_copy(idx_hbm.at[row_of(0)], idx_b[0])
  issue_gather(0)
  issue_idx(1, 1)

  def outer(r, carry):
    for u in range(3):
      it = r * 3 + u
      b, b1, b2 = u, (u + 1) % 3, (u + 2) % 3

      @pl.when(it + 2 < cpw)
      def _():
        issue_idx(it + 2, b2)

      @pl.when(it + 1 < cpw)
      def _():
        wait_idx(b1)
        issue_gather(b1)

      wait_gather(b)
      compute(b)
      pltpu.sync_copy(stage_b[b], acc_sh.at[idx_b[b].at[1]], add=True)
    return carry

  lax.fori_loop(0, cpw // 3, outer, 0)


def _edge_pass_l1(idxcat, packed, adst_tab, zinit):
  mesh = plsc.VectorSubcoreMesh(core_axis_name="c", subcore_axis_name="s")

  @functools.partial(
      pl.kernel,
      out_type=jax.ShapeDtypeStruct((2, NP, A1), jnp.float32),
      mesh=mesh,
      scratch_types=[
          [pltpu.VMEM((3, K), jnp.int32) for _ in range(3)],
          [pltpu.VMEM((K, D1), jnp.float32) for _ in range(3)],
          [pltpu.VMEM((K, 16), jnp.float32) for _ in range(3)],
          [pltpu.VMEM((K, A1), jnp.float32) for _ in range(3)],
          pltpu.VMEM_SHARED((NP, A1), jnp.float32),
          [pltpu.SemaphoreType.DMA for _ in range(3)],
          [pltpu.SemaphoreType.DMA for _ in range(3)],
          [pltpu.SemaphoreType.DMA for _ in range(3)],
      ],
      compiler_params=_SC_PARAMS,
  )
  def kern(idx_hbm, packed_hbm, adst_hbm, zero_hbm, out_hbm,
           idx_b, rows_b, arows_b, stage_b, acc_sh, isems, gsems, asems):
    cid = lax.axis_index("c")
    sid = lax.axis_index("s")
    rpw = NP // 16
    pltpu.sync_copy(zero_hbm.at[pl.ds(sid * rpw, rpw)],
                    acc_sh.at[pl.ds(sid * rpw, rpw)])
    plsc.subcore_barrier()
    iota = lax.iota(jnp.int32, 16)
    lane_lt4 = iota < 4
    row0 = cid * NCHUNKP + sid

    def compute(b):
      rows_v = rows_b[b]
      arows_v = arows_b[b]
      stage_v = stage_b[b]

      def group(g, c2):
        for k in range(16):
          e = g * 16 + k
          adstv = arows_v[e, pl.ds(0, 16)]
          asrcv = rows_v[e, pl.ds(0, 16)]
          z = asrcv + adstv
          z = jnp.maximum(z, z * 0.2)
          w = jnp.where(lane_lt4, jnp.exp(z), 0.0)
          stage_v[e, pl.ds(0, 16)] = w
          for j in range(4):
            bj = jnp.take_along_axis(
                w, jnp.full((16,), j, jnp.int32), axis=0,
                mode="promise_in_bounds")
            stage_v[e, pl.ds(8 + j * 16, 16)] = (
                rows_v[e, pl.ds(16 + j * 16, 16)] * bj)
        return c2

      lax.fori_loop(0, K // 16, group, 0)

    gathers = [(packed_hbm, 0, rows_b, gsems), (adst_hbm, 2, arows_b, asems)]
    _ring(idx_hbm, gathers, acc_sh, idx_b, stage_b,
          isems, CPW1, lambda it: row0 + 16 * it, compute)

    plsc.subcore_barrier()
    pltpu.sync_copy(acc_sh.at[pl.ds(sid * rpw, rpw)],
                    out_hbm.at[cid, pl.ds(sid * rpw, rpw)])

  return kern(idxcat, packed, adst_tab, zinit)


def _edge_pass_l2(idxcat, packed, adst_tab, zinit):
  mesh = plsc.VectorSubcoreMesh(core_axis_name="c", subcore_axis_name="s")

  @functools.partial(
      pl.kernel,
      out_type=jax.ShapeDtypeStruct((2, NP, D2), jnp.float32),
      mesh=mesh,
      scratch_types=[
          [pltpu.VMEM((3, K), jnp.int32) for _ in range(3)],
          [pltpu.VMEM((K, D2), jnp.float32) for _ in range(3)],
          [pltpu.VMEM((K, D2), jnp.float32) for _ in range(3)],
          pltpu.VMEM((TAB2,), jnp.float32),
          pltpu.VMEM_SHARED((NP, D2), jnp.float32),
          [pltpu.SemaphoreType.DMA for _ in range(3)],
          [pltpu.SemaphoreType.DMA for _ in range(3)],
      ],
      compiler_params=_SC_PARAMS,
  )
  def kern(idx_hbm, packed_hbm, adst_hbm, zero_hbm, out_hbm,
           idx_b, rows_b, stage_b, tab_v, acc_sh, isems, gsems):
    cid = lax.axis_index("c")
    sid = lax.axis_index("s")
    wid = sid * 2 + cid
    rpw = NP // 16
    pltpu.sync_copy(zero_hbm.at[pl.ds(sid * rpw, rpw)],
                    acc_sh.at[pl.ds(sid * rpw, rpw)])
    pltpu.sync_copy(adst_hbm, tab_v)
    plsc.subcore_barrier()
    iota = lax.iota(jnp.int32, 16)

    def compute(b):
      rows_v = rows_b[b]
      stage_v = stage_b[b]
      idx_v = idx_b[b]

      def group(g, c2):
        e16 = g * 16 + iota
        dstg = idx_v[1, pl.ds(g * 16, 16)]
        adstv = plsc.load_gather(tab_v, [dstg])
        asrcv = plsc.load_gather(
            rows_v, [e16, jnp.full((16,), 16, jnp.int32)])
        z = asrcv + adstv
        z = jnp.maximum(z, z * 0.2)
        w = jnp.exp(z)
        plsc.store_scatter(
            stage_v, [e16, jnp.full((16,), 16, jnp.int32)], w)
        for k in range(16):
          e = g * 16 + k
          bk = jnp.take_along_axis(
              w, jnp.full((16,), k, jnp.int32), axis=0,
              mode="promise_in_bounds")
          stage_v[e, pl.ds(0, 16)] = rows_v[e, pl.ds(0, 16)] * bk
        return c2

      lax.fori_loop(0, K // 16, group, 0)

    gathers = [(packed_hbm, 0, rows_b, gsems)]
    _ring(idx_hbm, gathers, acc_sh, idx_b, stage_b,
          isems, CPW2, lambda it: wid + 32 * it, compute)

    plsc.subcore_barrier()
    pltpu.sync_copy(acc_sh.at[pl.ds(sid * rpw, rpw)],
                    out_hbm.at[cid, pl.ds(sid * rpw, rpw)])

  return kern(idxcat, packed, adst_tab, zinit)


def _prep1(x, W1, Asrc, Adst, R8):
  def body(x_ref, w1_ref, as_ref, ad_ref, r8_ref,
           packed_ref, adst_ref, self_ref):
    h = jnp.dot(x_ref[...], w1_ref[...], preferred_element_type=jnp.float32)
    asrc = jnp.dot(h, as_ref[...], preferred_element_type=jnp.float32)
    adst = jnp.dot(h, ad_ref[...], preferred_element_type=jnp.float32)
    z = asrc + adst
    w = jnp.exp(jnp.maximum(z, 0.2 * z))
    wrep = jnp.dot(w, r8_ref[...], preferred_element_type=jnp.float32)
    zpad = jnp.zeros((h.shape[0], 12), jnp.float32)
    for q in range(2):
      packed_ref[q, :, 0:4] = asrc[:, 4 * q:4 * q + 4]
      packed_ref[q, :, 4:16] = zpad
      packed_ref[q, :, 16:80] = h[:, 64 * q:64 * q + 64]
    self_ref[:, 0:128] = h * wrep
    self_ref[:, 128:136] = w
    self_ref[:, 136:144] = jnp.zeros_like(w)
    adst_ref[...] = adst

  return pl.pallas_call(
      body,
      out_shape=[
          jax.ShapeDtypeStruct((2, N, D1), jnp.float32),
          jax.ShapeDtypeStruct((N, HEADS), jnp.float32),
          jax.ShapeDtypeStruct((N, 144), jnp.float32),
      ],
  )(x, W1, Asrc, Adst, R8)


def _prep2(p0, p1, si1, b1, W2, att_s2, att_d2, R8):
  def body(p0_ref, p1_ref, si_ref, b1_ref, w2_ref, as_ref, ad_ref, r8_ref,
           packed_ref, adst_ref, self_ref):
    p0 = p0_ref[...]
    p1 = p1_ref[...]
    si = si_ref[...]
    m = jnp.concatenate([p0[:, 8:72], p1[:, 8:72]], axis=1) + si[:, 0:128]
    s = jnp.concatenate([p0[:, 0:4], p1[:, 0:4]], axis=1) + si[:, 128:136]
    srep = jnp.dot(s, r8_ref[...], preferred_element_type=jnp.float32)
    o1 = m / (srep + 1e-16) + b1_ref[...]
    e1 = jnp.where(o1 > 0, o1, jnp.exp(o1) - 1.0)
    h2 = jnp.dot(e1, w2_ref[...], preferred_element_type=jnp.float32)
    as2 = jnp.sum(h2 * as_ref[...], axis=1, keepdims=True)
    ad2 = jnp.sum(h2 * ad_ref[...], axis=1, keepdims=True)
    z = as2 + ad2
    w = jnp.exp(jnp.maximum(z, 0.2 * z))
    zpad = jnp.zeros((h2.shape[0], 15), jnp.float32)
    packed_ref[:, 0:16] = h2
    packed_ref[:, 16:17] = as2
    packed_ref[:, 17:32] = zpad
    self_ref[:, 0:16] = h2 * w
    self_ref[:, 16:17] = w
    self_ref[:, 17:32] = zpad
    adst_ref[...] = ad2

  return pl.pallas_call(
      body,
      out_shape=[
          jax.ShapeDtypeStruct((N, D2), jnp.float32),
          jax.ShapeDtypeStruct((N, 1), jnp.float32),
          jax.ShapeDtypeStruct((N, D2), jnp.float32),
      ],
  )(p0, p1, si1, b1, W2, att_s2, att_d2, R8)


def _final(q0, q1, si2, b2):
  def body(q0_ref, q1_ref, si_ref, b2_ref, out_ref):
    acc = q0_ref[...] + q1_ref[...] + si_ref[...]
    o = acc[:, 0:16] / (acc[:, 16:17] + 1e-16) + b2_ref[...]
    mx = jnp.max(o, axis=1, keepdims=True)
    lse = jnp.log(jnp.sum(jnp.exp(o - mx), axis=1, keepdims=True))
    out_ref[...] = o - mx - lse

  return pl.pallas_call(
      body,
      out_shape=jax.ShapeDtypeStruct((N, OUT_SIZE), jnp.float32),
  )(q0, q1, si2, b2)


def kernel(x, edge_index, W1, att_src1, att_dst1, b1, W2, att_src2, att_dst2,
           b2):
  src = jnp.pad(edge_index[0].astype(jnp.int32), (0, EPAD - E))
  dst = jnp.pad(edge_index[1].astype(jnp.int32), (0, EPAD - E),
                constant_values=N)
  s0 = src.reshape(NCHUNKP, 1, K)
  s1 = (src + N).reshape(NCHUNKP, 1, K)
  dd = dst.reshape(NCHUNKP, 1, K)
  d1r = (dst + NPAD2).reshape(NCHUNKP, 1, K)
  idxcat = jnp.concatenate([
      jnp.concatenate([s0, dd, dd], axis=1),
      jnp.concatenate([s1, dd, d1r], axis=1)], axis=0)

  eye8 = jnp.eye(HEADS, dtype=jnp.float32)
  Asrc = (att_src1[:, :, None] * eye8[:, None, :]).reshape(IN_SIZE, HEADS)
  Adst = (att_dst1[:, :, None] * eye8[:, None, :]).reshape(IN_SIZE, HEADS)
  R8 = jnp.kron(eye8, jnp.ones((1, HID), jnp.float32))

  packed1, adst1, si1 = _prep1(x, W1, Asrc, Adst, R8)
  tab1 = jnp.concatenate([
      jnp.pad(adst1[:, 0:4], ((0, NPAD2 - N), (0, 12))),
      jnp.pad(adst1[:, 4:8], ((0, NPAD2 - N), (0, 12))),
  ])
  z1 = jnp.zeros((NP, A1), jnp.float32)
  acc1 = _edge_pass_l1(idxcat, packed1.reshape(2 * N, D1), tab1, z1)

  packed2, adst2, si2 = _prep2(
      acc1[0, :N], acc1[1, :N], si1, b1.reshape(1, IN_SIZE), W2,
      att_src2, att_dst2, R8)
  tab2 = jnp.pad(adst2.reshape(-1), (0, TAB2 - N))
  z2 = jnp.zeros((NP, D2), jnp.float32)
  acc2 = _edge_pass_l2(idxcat, packed2, tab2, z2)

  return _final(acc2[0, :N], acc2[1, :N], si2, b2.reshape(1, OUT_SIZE))
